# Initial kernel scaffold; baseline (speedup 1.0000x reference)
import jax
import jax.numpy as jnp
from jax.experimental import pallas as pl


def kernel(edge_index, node_feat, edge_feat, edge_embed, dim_size, fc_w0, fc_w1, fc_w2, sc_w):
    raise NotImplementedError("write your pallas kernel here")


# speedup vs baseline: 3.0288x; 3.0288x over previous
import functools

import jax
import jax.numpy as jnp
import numpy as np
from jax import lax
from jax.experimental import pallas as pl
from jax.experimental.pallas import tpu as pltpu
from jax.experimental.pallas import tpu_sc as plsc

_rng = np.random.RandomState(12345)
_xs = _rng.randn(1000000)
_SILU_CST = float(1.0 / np.sqrt(np.mean((_xs / (1.0 + np.exp(-_xs))) ** 2)))


def _msg_body(eb_ref, ef_ref, xs_ref, w0_ref, w1_ref, w2_ref, out_ref):
    h = eb_ref[...] @ w0_ref[...]
    h = _SILU_CST * (h * jax.nn.sigmoid(h))
    h = h @ w1_ref[...]
    h = _SILU_CST * (h * jax.nn.sigmoid(h))
    wp = h @ w2_ref[...]
    ef = ef_ref[...]
    acc = wp[:, 0:128] * ef[:, 0:1]
    acc += wp[:, 128:256] * ef[:, 1:2]
    acc += wp[:, 256:384] * ef[:, 2:3]
    acc += wp[:, 384:512] * ef[:, 3:4]
    out_ref[...] = acc * xs_ref[...]


def _make_msg_call(E, BE, RAD_EMBED, MUL_IN, MUL_EDGE):
    grid = (E // BE,)
    return pl.pallas_call(
        _msg_body,
        grid=grid,
        in_specs=[
            pl.BlockSpec((BE, RAD_EMBED), lambda i: (i, 0)),
            pl.BlockSpec((BE, MUL_EDGE), lambda i: (i, 0)),
            pl.BlockSpec((BE, MUL_IN), lambda i: (i, 0)),
            pl.BlockSpec((RAD_EMBED, 128), lambda i: (0, 0)),
            pl.BlockSpec((128, 128), lambda i: (0, 0)),
            pl.BlockSpec((128, MUL_EDGE * MUL_IN), lambda i: (0, 0)),
        ],
        out_specs=pl.BlockSpec((BE, MUL_IN), lambda i: (i, 0)),
        out_shape=jax.ShapeDtypeStruct((E, MUL_IN), jnp.float32),
    )


def _final_body(p_ref, nf_ref, w_ref, out_ref):
    out_ref[...] = p_ref[...] + nf_ref[...] @ w_ref[...]


def _make_final_call(N, BN, MUL_IN):
    return pl.pallas_call(
        _final_body,
        grid=(N // BN,),
        in_specs=[
            pl.BlockSpec((BN, MUL_IN), lambda i: (i, 0)),
            pl.BlockSpec((BN, MUL_IN), lambda i: (i, 0)),
            pl.BlockSpec((MUL_IN, MUL_IN), lambda i: (0, 0)),
        ],
        out_specs=pl.BlockSpec((BN, MUL_IN), lambda i: (i, 0)),
        out_shape=jax.ShapeDtypeStruct((N, MUL_IN), jnp.float32),
    )


def _make_gather(E, N, D, NC, NS, C):
    NW = NC * NS
    per_w = E // NW
    n_chunks = per_w // C
    mesh = plsc.VectorSubcoreMesh(core_axis_name="c", subcore_axis_name="s")

    @functools.partial(
        pl.kernel,
        out_type=jax.ShapeDtypeStruct((E, D), jnp.float32),
        mesh=mesh,
        scratch_types=[
            pltpu.VMEM((C,), jnp.int32),
            pltpu.VMEM((C, D), jnp.float32),
            pltpu.SemaphoreType.DMA,
        ],
    )
    def gather_k(nf_hbm, src_hbm, xsrc_hbm, idx_v, rows_v, sem):
        wid = lax.axis_index("s") * NC + lax.axis_index("c")
        base = wid * per_w

        @pl.loop(0, n_chunks)
        def _chunk(i):
            off = base + i * C
            pltpu.sync_copy(src_hbm.at[pl.ds(off, C)], idx_v)
            pltpu.async_copy(nf_hbm.at[idx_v], rows_v, sem).wait()
            pltpu.sync_copy(rows_v, xsrc_hbm.at[pl.ds(off, C)])

    return gather_k


def _make_scatter(E, N, D, NC, NS, C, ZR):
    half = N // 2
    per_tile = E // NS
    n_chunks = per_tile // C
    n_zchunks = half // ZR
    mesh = plsc.VectorSubcoreMesh(core_axis_name="c", subcore_axis_name="s")

    @functools.partial(
        pl.kernel,
        out_type=jax.ShapeDtypeStruct((N, D), jnp.float32),
        mesh=mesh,
        scratch_types=[
            pltpu.VMEM((C,), jnp.int32),
            pltpu.VMEM((C, D), jnp.float32),
            pltpu.VMEM((ZR, D), jnp.float32),
            pltpu.VMEM_SHARED((half + 8, D), jnp.float32),
            pltpu.SemaphoreType.DMA,
        ],
    )
    def scatter_k(msg_hbm, dst_hbm, zeros_hbm, out_hbm,
                  idx_v, rows_v, zbuf, acc_sh, sem):
        cid = lax.axis_index("c")
        sid = lax.axis_index("s")
        my_lo = cid * half
        base = sid * per_tile

        pltpu.sync_copy(zeros_hbm.at[pl.ds(0, ZR)], zbuf)

        @pl.loop(sid, n_zchunks, step=NS)
        def _z(j):
            pltpu.sync_copy(zbuf, acc_sh.at[pl.ds(j * ZR, ZR)])

        plsc.subcore_barrier()

        lanes = lax.iota(jnp.int32, 16)

        @pl.loop(0, n_chunks)
        def _chunk(i):
            off = base + i * C
            pltpu.sync_copy(dst_hbm.at[pl.ds(off, C)], idx_v)
            pltpu.sync_copy(msg_hbm.at[pl.ds(off, C)], rows_v)
            for k in range(C // 16):
                t = idx_v[pl.ds(k * 16, 16)] - my_lo
                inr = (t >= 0) & (t < half)
                idx_v[pl.ds(k * 16, 16)] = jnp.where(
                    inr, t, half + (lanes & 7))
            pltpu.sync_copy(rows_v, acc_sh.at[idx_v], add=True)

        plsc.subcore_barrier()

        @pl.loop(sid, n_zchunks, step=NS)
        def _wb(j):
            pltpu.sync_copy(acc_sh.at[pl.ds(j * ZR, ZR)], zbuf)
            pltpu.sync_copy(zbuf, out_hbm.at[pl.ds(my_lo + j * ZR, ZR)])

    return scatter_k


def kernel(edge_index, node_feat, edge_feat, edge_embed, dim_size,
           fc_w0, fc_w1, fc_w2, sc_w):
    E = edge_index.shape[1]
    N, MUL_IN = node_feat.shape
    MUL_EDGE = edge_feat.shape[1]
    RAD_EMBED = edge_embed.shape[1]
    path_weight = 1.0 / np.sqrt(float(MUL_EDGE))

    src = edge_index[0]
    dst = edge_index[1]

    w0n = fc_w0 / np.sqrt(float(fc_w0.shape[0]))
    w1n = fc_w1 / np.sqrt(float(fc_w1.shape[0]))
    w2n = (fc_w2 * (path_weight / np.sqrt(float(fc_w2.shape[0]))))
    w2p = w2n.reshape(fc_w2.shape[0], MUL_IN, MUL_EDGE).transpose(0, 2, 1) \
             .reshape(fc_w2.shape[0], MUL_EDGE * MUL_IN)
    sc_wn = sc_w / np.sqrt(float(MUL_IN))

    info = plsc.get_sparse_core_info()
    NC, NS = info.num_cores, info.num_subcores

    gather_k = _make_gather(E, N, MUL_IN, NC, NS, C=80)
    x_src = gather_k(node_feat, src)

    msg_call = _make_msg_call(E, 2560, RAD_EMBED, MUL_IN, MUL_EDGE)
    msg = msg_call(edge_embed, edge_feat, x_src, w0n, w1n, w2p)

    zeros = jnp.zeros((200, MUL_IN), jnp.float32)
    scatter_k = _make_scatter(E, N, MUL_IN, NC, NS, C=80, ZR=200)
    scat = scatter_k(msg, dst, zeros)

    final_call = _make_final_call(N, 2000, MUL_IN)
    return final_call(scat, node_feat, sc_wn)
